# Initial kernel scaffold; baseline (speedup 1.0000x reference)
import jax
import jax.numpy as jnp
from jax.experimental import pallas as pl


def kernel(q, c_t, W_a, W_p):
    raise NotImplementedError("write your pallas kernel here")


# speedup vs baseline: 1.5324x; 1.5324x over previous
import functools

import jax
import jax.numpy as jnp
from jax.experimental import pallas as pl


def _attn_body(q_ref, wa16_ref, ct_ref, pt_ref, out_ref, *, hh, ww, half):
    qb = q_ref[0]
    qb16 = qb.astype(jnp.bfloat16)
    ct16 = ct_ref[0].astype(jnp.bfloat16)
    p0 = pt_ref[0, 0, 0]
    p1 = pt_ref[0, 0, 1]
    p0r = jnp.round(p0)
    p1r = jnp.round(p1)

    yT = jax.lax.dot_general(
        wa16_ref[...], qb16, (((1,), (0,)), ((), ())),
        preferred_element_type=jnp.float32)
    yT16 = yT.astype(jnp.bfloat16)
    score = jax.lax.dot_general(
        ct16, yT16, (((1,), (0,)), ((), ())),
        preferred_element_type=jnp.float32)

    i = jax.lax.broadcasted_iota(jnp.int32, (1, hh * ww), 1).astype(jnp.float32)
    hf = jnp.floor((i + 0.5) * (1.0 / ww))
    wf = i - ww * hf
    inv = 1.0 / half
    shift = 2.0 * ((hf - p0) * inv) ** 2 + 2.0 * ((wf - p1) * inv) ** 2
    mask = ((hf >= p0r - half) & (hf <= p0r + (half - 1))
            & (wf >= p1r - half) & (wf <= p1r + (half - 1)))

    a = jnp.where(mask, score - shift, -jnp.inf)
    m = jnp.max(a, axis=1, keepdims=True)
    e = jnp.where(mask, jnp.exp(a - m), 0.0)
    wts = e / jnp.sum(e, axis=1, keepdims=True)

    out_ref[0] = jnp.sum(qb * wts, axis=1, keepdims=True)


def kernel(q, c_t, W_a, W_p):
    B, Q, H, W = q.shape
    C = c_t.shape[1]
    q3 = q.reshape(B, Q, H * W)

    pt = H * jax.nn.sigmoid(c_t @ W_p.T)
    pt3 = pt.reshape(B, 1, 2)
    ct3 = c_t.reshape(B, 1, C)
    wa16 = W_a.astype(jnp.bfloat16)

    out3 = pl.pallas_call(
        functools.partial(_attn_body, hh=H, ww=W, half=8),
        grid=(B,),
        in_specs=[
            pl.BlockSpec((1, Q, H * W), lambda b: (b, 0, 0)),
            pl.BlockSpec((C, Q), lambda b: (0, 0)),
            pl.BlockSpec((1, 1, C), lambda b: (b, 0, 0)),
            pl.BlockSpec((1, 1, 2), lambda b: (b, 0, 0)),
        ],
        out_specs=pl.BlockSpec((1, Q, 1), lambda b: (b, 0, 0)),
        out_shape=jax.ShapeDtypeStruct((B, Q, 1), jnp.float32),
    )(q3, wa16, ct3, pt3)
    return out3.reshape(B, Q)


# speedup vs baseline: 1.5357x; 1.0022x over previous
import functools

import jax
import jax.numpy as jnp
from jax.experimental import pallas as pl

_NB = 4


def _attn_body(q_ref, wa16_ref, ct_ref, pt_ref, out_ref, *, hh, ww, half, nb):
    hw = hh * ww
    i = jax.lax.broadcasted_iota(jnp.int32, (1, hw), 1).astype(jnp.float32)
    hf = jnp.floor((i + 0.5) * (1.0 / ww))
    wf = i - ww * hf
    inv = 1.0 / half
    wa16 = wa16_ref[...]

    for k in range(nb):
        qb16 = q_ref[k]
        ct16 = ct_ref[k].astype(jnp.bfloat16)
        p0 = pt_ref[k, 0, 0]
        p1 = pt_ref[k, 0, 1]
        p0r = jnp.round(p0)
        p1r = jnp.round(p1)

        yT = jax.lax.dot_general(
            wa16, qb16, (((1,), (0,)), ((), ())),
            preferred_element_type=jnp.float32)
        yT16 = yT.astype(jnp.bfloat16)
        score = jax.lax.dot_general(
            ct16, yT16, (((1,), (0,)), ((), ())),
            preferred_element_type=jnp.float32)

        shift = (2.0 * ((hf - p0) * inv) ** 2
                 + 2.0 * ((wf - p1) * inv) ** 2)
        mask = ((hf >= p0r - half) & (hf <= p0r + (half - 1))
                & (wf >= p1r - half) & (wf <= p1r + (half - 1)))

        a = jnp.where(mask, score - shift, -jnp.inf)
        m = jnp.max(a, axis=1, keepdims=True)
        e = jnp.where(mask, jnp.exp(a - m), 0.0)
        wts = e / jnp.sum(e, axis=1, keepdims=True)

        out_ref[k] = jnp.sum(qb16.astype(jnp.float32) * wts,
                             axis=1, keepdims=True)


def kernel(q, c_t, W_a, W_p):
    B, Q, H, W = q.shape
    C = c_t.shape[1]
    qbf = q.reshape(B, Q, H * W).astype(jnp.bfloat16)

    pt = H * jax.nn.sigmoid(c_t @ W_p.T)
    pt3 = pt.reshape(B, 1, 2)
    ct3 = c_t.reshape(B, 1, C)
    wa16 = W_a.astype(jnp.bfloat16)

    nsteps = B // _NB
    out3 = pl.pallas_call(
        functools.partial(_attn_body, hh=H, ww=W, half=8, nb=_NB),
        grid=(nsteps,),
        in_specs=[
            pl.BlockSpec((_NB, Q, H * W), lambda b: (b, 0, 0)),
            pl.BlockSpec((C, Q), lambda b: (0, 0)),
            pl.BlockSpec((_NB, 1, C), lambda b: (b, 0, 0)),
            pl.BlockSpec((_NB, 1, 2), lambda b: (b, 0, 0)),
        ],
        out_specs=pl.BlockSpec((_NB, Q, 1), lambda b: (b, 0, 0)),
        out_shape=jax.ShapeDtypeStruct((B, Q, 1), jnp.float32),
    )(qbf, wa16, ct3, pt3)
    return out3.reshape(B, Q)


# speedup vs baseline: 1.7312x; 1.1273x over previous
import functools

import jax
import jax.numpy as jnp
from jax.experimental import pallas as pl

_NB = 4


def _prep_body(ct_ref, wa_ref, u_ref):
    ct16 = ct_ref[...].astype(jnp.bfloat16)
    wa16 = wa_ref[...].astype(jnp.bfloat16)
    u_ref[...] = jnp.dot(ct16, wa16, preferred_element_type=jnp.float32)


def _attn_body(q_ref, u_ref, pt_ref, out_ref, *, hh, ww, half, nb):
    hw = hh * ww
    i = jax.lax.broadcasted_iota(jnp.int32, (1, hw), 1).astype(jnp.float32)
    hf = jnp.floor((i + 0.5) * (1.0 / ww))
    wf = i - ww * hf
    inv = 1.0 / half

    for k in range(nb):
        qbf = q_ref[k].astype(jnp.float32)
        ucol = u_ref[k]
        p0 = pt_ref[k, 0, 0]
        p1 = pt_ref[k, 0, 1]
        p0r = jnp.round(p0)
        p1r = jnp.round(p1)

        score = jnp.sum(qbf * ucol, axis=0, keepdims=True)

        shift = (2.0 * ((hf - p0) * inv) ** 2
                 + 2.0 * ((wf - p1) * inv) ** 2)
        mask = ((hf >= p0r - half) & (hf <= p0r + (half - 1))
                & (wf >= p1r - half) & (wf <= p1r + (half - 1)))

        a = jnp.where(mask, score - shift, -jnp.inf)
        m = jnp.max(a, axis=1, keepdims=True)
        e = jnp.where(mask, jnp.exp(a - m), 0.0)
        wts = e / jnp.sum(e, axis=1, keepdims=True)

        out_ref[k] = jnp.sum(qbf * wts, axis=1, keepdims=True)


def kernel(q, c_t, W_a, W_p):
    B, Q, H, W = q.shape
    C = c_t.shape[1]
    qbf = q.astype(jnp.bfloat16).reshape(B, Q, H * W)

    pt = H * jax.nn.sigmoid(c_t @ W_p.T)
    pt3 = pt.reshape(B, 1, 2)

    u = pl.pallas_call(
        _prep_body,
        out_shape=jax.ShapeDtypeStruct((B, Q), jnp.float32),
    )(c_t, W_a)
    u3 = u.reshape(B, Q, 1)

    nsteps = B // _NB
    out3 = pl.pallas_call(
        functools.partial(_attn_body, hh=H, ww=W, half=8, nb=_NB),
        grid=(nsteps,),
        in_specs=[
            pl.BlockSpec((_NB, Q, H * W), lambda b: (b, 0, 0)),
            pl.BlockSpec((_NB, Q, 1), lambda b: (b, 0, 0)),
            pl.BlockSpec((_NB, 1, 2), lambda b: (b, 0, 0)),
        ],
        out_specs=pl.BlockSpec((_NB, Q, 1), lambda b: (b, 0, 0)),
        out_shape=jax.ShapeDtypeStruct((B, Q, 1), jnp.float32),
    )(qbf, u3, pt3)
    return out3.reshape(B, Q)
